# baseline (speedup 1.0000x reference)
import dataclasses

import jax
import jax.numpy as jnp
from jax import lax
from jax.experimental import pallas as pl
from jax.experimental.pallas import tpu as pltpu
from jax.experimental.pallas import tpu_sc as plsc

_B = 16384
_F = 26
_K = 16
_FEAT = 100000
_ROWS = _FEAT * _F
_WR = _ROWS // 16
_NC = 2
_NS = 16
_NW = _NC * _NS
_EPW = _B // _NW
_C = 64
_NCH = _EPW // _C
_IPC = _C * _F
_IROWS = _IPC // 128
_G = _C // 16


def _fm_body(idx_hbm, val_hbm, offs_hbm, w_hbm, v_hbm, out_hbm,
             idx0, idx1, lanev, offsv, valv, cbuf, vbuf, rowsA, rowsB,
             sbuf, fbuf, outbuf, semA, semB):
    wid = lax.axis_index("s") * _NC + lax.axis_index("c")
    pltpu.sync_copy(offs_hbm, offsv)

    iota = lax.iota(jnp.int32, 16)
    iota_f16 = iota * (_F * 16)
    zeros_i = jnp.zeros((16,), jnp.int32)
    zero = jnp.zeros((16,), jnp.float32)

    def fire(table, idxr, rows, sem):
        return [
            pltpu.async_copy(table.at[idxr.at[pl.ds(j * 128, 128)]],
                             rows.at[pl.ds(j * 128, 128)], sem)
            for j in range(_IROWS)
        ]

    def bump(dst, srcr):
        @pl.loop(0, _IPC // 16)
        def _b(j):
            sl = pl.ds(j * 16, 16)
            dst[sl] = srcr[sl] + _WRV

    def compute(rows, is_w):
        @pl.loop(0, _G)
        def _grp(g):
            osl = pl.ds(g * 16, 16)
            if is_w:
                facc = zero
                for f in range(_F):
                    sl = pl.ds((g * _F + f) * 16, 16)
                    wv = plsc.load_gather(rows, [zeros_i, cbuf[sl]])
                    facc = facc + wv * vbuf[sl]
                fbuf[osl] = facc
                sbuf[osl] = zero
            else:
                acc = zero
                acc2 = zero
                for f in range(_F):
                    sl = pl.ds((g * _F + f) * 16, 16)
                    gv = plsc.load_gather(rows, [zeros_i, cbuf[sl]])
                    rv = gv * vbuf[sl]
                    acc = acc + rv
                    acc2 = acc2 + rv * rv
                sbuf[osl] = sbuf[osl] + (acc * acc - acc2)

    @pl.loop(0, _NCH)
    def _chunk(ch):
        ebase = wid * _EPW + ch * _C
        pltpu.sync_copy(idx_hbm.at[pl.ds(ebase * _F, _IPC)], idx0)
        pltpu.sync_copy(val_hbm.at[pl.ds(ebase * _F, _IPC)], valv)

        @pl.loop(0, _IPC // 16)
        def _off(j):
            sl = pl.ds(j * 16, 16)
            full = idx0[sl] + offsv[sl]
            lanev[sl] = full & 15
            idx0[sl] = lax.shift_right_logical(full, 4)

        @pl.loop(0, _G)
        def _pre(g):
            for f in range(_F):
                pos = (iota * _F) + (g * 16 * _F + f)
                sl = pl.ds((g * _F + f) * 16, 16)
                cbuf[sl] = iota_f16 + ((g * 16 * _F + f) * 16
                                       + plsc.load_gather(lanev, [pos]))
                vbuf[sl] = plsc.load_gather(valv, [pos])

        idxs = [idx0, idx1]
        rows = [rowsA, rowsB]
        sems = [semA, semB]
        pend1 = fire(w_hbm, idx0, rowsA, semA)
        pend2 = fire(v_hbm, idx0, rowsB, semB)
        bump(idx1, idx0)
        pending = [pend1, pend2]
        for p in range(17):
            for cp_ in pending[p]:
                cp_.wait()
            if 1 <= p < 16:
                nxt = fire(v_hbm, idxs[p % 2], rows[(p + 1) % 2],
                           sems[(p + 1) % 2])
                pending.append(nxt)
                bump(idxs[(p + 1) % 2], idxs[p % 2])
            compute(rows[p % 2], is_w=(p == 0))

        @pl.loop(0, _G)
        def _out(g):
            osl = pl.ds(g * 16, 16)
            outbuf[pl.ds(ch * _C + g * 16, 16)] = fbuf[osl] + 0.5 * sbuf[osl]

    pltpu.sync_copy(outbuf, out_hbm.at[pl.ds(wid * _EPW, _EPW)])


_DCH = 131072
_DNB = 20
_VPAD = _DCH * _DNB
_WRV = _VPAD // 16


def _detile_body(i_ref, o_ref):
    o_ref[...] = i_ref[pl.program_id(2), :]


def _detile(vT):
    return pl.pallas_call(
        _detile_body,
        grid=(2, _DNB, 8),
        in_specs=[pl.BlockSpec((8, _DCH), lambda kb, c, kp: (kb, c))],
        out_specs=pl.BlockSpec((_DCH,),
                               lambda kb, c, kp: ((kb * 8 + kp) * _DNB + c,)),
        out_shape=jax.ShapeDtypeStruct((_K * _VPAD,), jnp.float32),
    )(vT)


def _wcopy_body(w_ref, o_ref, sem):
    pltpu.async_copy(w_ref, o_ref, sem).wait()


def _wcopy(w):
    return pl.pallas_call(
        _wcopy_body,
        in_specs=[pl.BlockSpec(memory_space=pl.ANY)],
        out_specs=pl.BlockSpec(memory_space=pl.ANY),
        out_shape=jax.ShapeDtypeStruct((_ROWS, 1), jnp.float32),
        scratch_shapes=[pltpu.SemaphoreType.DMA],
    )(w)


def kernel(inputs_index, inputs_value, w0, w, V):
    idxflat = inputs_index.astype(jnp.int32).reshape(_B * _F)
    valflat = inputs_value.reshape(_B * _F)
    offs = jnp.tile(jnp.arange(_F, dtype=jnp.int32) * _FEAT, _C)
    vt = _detile(jnp.swapaxes(V, 0, 1)).reshape(_K * _WRV, 16)
    wt = _wcopy(w).reshape(_WR, 16)

    mesh = plsc.VectorSubcoreMesh(core_axis_name="c", subcore_axis_name="s",
                                  num_cores=_NC, num_subcores=_NS)
    cp = pltpu.CompilerParams()
    if "needs_layout_passes" in pltpu.CompilerParams.__dataclass_fields__:
        cp = dataclasses.replace(cp, needs_layout_passes=False)
    cp = dataclasses.replace(cp, use_tc_tiling_on_sc=False)
    fm = pl.kernel(
        _fm_body,
        out_type=jax.ShapeDtypeStruct((_B,), jnp.float32),
        mesh=mesh,
        scratch_types=[
            pltpu.VMEM((_IPC,), jnp.int32),
            pltpu.VMEM((_IPC,), jnp.int32),
            pltpu.VMEM((_IPC,), jnp.int32),
            pltpu.VMEM((_IPC,), jnp.int32),
            pltpu.VMEM((_IPC,), jnp.float32),
            pltpu.VMEM((_IPC,), jnp.int32),
            pltpu.VMEM((_IPC,), jnp.float32),
            pltpu.VMEM((_IPC, 16), jnp.float32),
            pltpu.VMEM((_IPC, 16), jnp.float32),
            pltpu.VMEM((_G * 16,), jnp.float32),
            pltpu.VMEM((_G * 16,), jnp.float32),
            pltpu.VMEM((_EPW,), jnp.float32),
            pltpu.SemaphoreType.DMA,
            pltpu.SemaphoreType.DMA,
        ],
        compiler_params=cp,
    )
    out = fm(idxflat, valflat, offs, wt, vt)
    return out.reshape(_B, 1) + w0


# speedup vs baseline: 58.1157x; 58.1157x over previous
import dataclasses

import jax
import jax.numpy as jnp
from jax import lax
from jax.experimental import pallas as pl
from jax.experimental.pallas import tpu as pltpu
from jax.experimental.pallas import tpu_sc as plsc

_B = 16384
_F = 26
_K = 16
_FEAT = 100000
_ROWS = _FEAT * _F
_WR = _ROWS // 16
_NC = 2
_NS = 16
_NW = _NC * _NS
_EPW = _B // _NW
_C = 128
_NCH = _EPW // _C
_IPC = _C * _F
_IROWS = _IPC // 128
_G = _C // 16


def _fm_body(idx_hbm, val_hbm, offs_hbm, w_hbm, v_hbm, out_hbm,
             idx0, idx1, lanev, offsv, valv, cbuf, vbuf, rowsA, rowsB,
             sbuf, fbuf, outbuf, semA, semB):
    wid = lax.axis_index("s") * _NC + lax.axis_index("c")
    pltpu.sync_copy(offs_hbm, offsv)

    iota = lax.iota(jnp.int32, 16)
    iota_f16 = iota * (_F * 16)
    zeros_i = jnp.zeros((16,), jnp.int32)
    zero = jnp.zeros((16,), jnp.float32)

    def fire(table, idxr, rows, sem):
        return [
            pltpu.async_copy(table.at[idxr.at[pl.ds(j * 128, 128)]],
                             rows.at[pl.ds(j * 128, 128)], sem)
            for j in range(_IROWS)
        ]

    def bump(dst, srcr):
        @pl.loop(0, _IPC // 16)
        def _b(j):
            sl = pl.ds(j * 16, 16)
            dst[sl] = srcr[sl] + _WRV

    def compute(rows, is_w):
        @pl.loop(0, _G)
        def _grp(g):
            osl = pl.ds(g * 16, 16)
            if is_w:
                facc = zero
                for f in range(_F):
                    sl = pl.ds((g * _F + f) * 16, 16)
                    wv = plsc.load_gather(rows, [zeros_i, cbuf[sl]])
                    facc = facc + wv * vbuf[sl]
                fbuf[osl] = facc
                sbuf[osl] = zero
            else:
                acc = zero
                acc2 = zero
                for f in range(_F):
                    sl = pl.ds((g * _F + f) * 16, 16)
                    gv = plsc.load_gather(rows, [zeros_i, cbuf[sl]])
                    rv = gv * vbuf[sl]
                    acc = acc + rv
                    acc2 = acc2 + rv * rv
                sbuf[osl] = sbuf[osl] + (acc * acc - acc2)

    @pl.loop(0, _NCH)
    def _chunk(ch):
        ebase = wid * _EPW + ch * _C
        pltpu.sync_copy(idx_hbm.at[pl.ds(ebase * _F, _IPC)], idx0)
        pltpu.sync_copy(val_hbm.at[pl.ds(ebase * _F, _IPC)], valv)

        @pl.loop(0, _IPC // 16)
        def _off(j):
            sl = pl.ds(j * 16, 16)
            full = idx0[sl] + offsv[pl.ds((j % 104) * 16, 16)]
            lanev[sl] = full & 15
            idx0[sl] = lax.shift_right_logical(full, 4)

        @pl.loop(0, _G)
        def _pre(g):
            for f in range(_F):
                pos = (iota * _F) + (g * 16 * _F + f)
                sl = pl.ds((g * _F + f) * 16, 16)
                cbuf[sl] = iota_f16 + ((g * 16 * _F + f) * 16
                                       + plsc.load_gather(lanev, [pos]))
                vbuf[sl] = plsc.load_gather(valv, [pos])

        idxs = [idx0, idx1]
        rows = [rowsA, rowsB]
        sems = [semA, semB]
        pend1 = fire(w_hbm, idx0, rowsA, semA)
        pend2 = fire(v_hbm, idx0, rowsB, semB)
        bump(idx1, idx0)
        pending = [pend1, pend2]
        for p in range(17):
            for cp_ in pending[p]:
                cp_.wait()
            if 1 <= p < 16:
                nxt = fire(v_hbm, idxs[p % 2], rows[(p + 1) % 2],
                           sems[(p + 1) % 2])
                pending.append(nxt)
                bump(idxs[(p + 1) % 2], idxs[p % 2])
            compute(rows[p % 2], is_w=(p == 0))

        @pl.loop(0, _G)
        def _out(g):
            osl = pl.ds(g * 16, 16)
            outbuf[pl.ds(ch * _C + g * 16, 16)] = fbuf[osl] + 0.5 * sbuf[osl]

    pltpu.sync_copy(outbuf, out_hbm.at[pl.ds(wid * _EPW, _EPW)])


_DCH = 131072
_DNB = 20
_VPAD = _DCH * _DNB
_WRV = _VPAD // 16


def _detile_body(i_ref, o_ref):
    o_ref[...] = i_ref[pl.program_id(2), :]


def _detile(vT):
    return pl.pallas_call(
        _detile_body,
        grid=(2, _DNB, 8),
        in_specs=[pl.BlockSpec((8, _DCH), lambda kb, c, kp: (kb, c))],
        out_specs=pl.BlockSpec((_DCH,),
                               lambda kb, c, kp: ((kb * 8 + kp) * _DNB + c,)),
        out_shape=jax.ShapeDtypeStruct((_K * _VPAD,), jnp.float32),
    )(vT)


def kernel(inputs_index, inputs_value, w0, w, V):
    idxflat = inputs_index.astype(jnp.int32).reshape(_B * _F)
    valflat = inputs_value.reshape(_B * _F)
    offs = jnp.tile(jnp.arange(_F, dtype=jnp.int32) * _FEAT, 64)
    vt = _detile(jnp.swapaxes(V, 0, 1)).reshape(_K * _WRV, 16)
    wt = w[:, 0].reshape(_WR, 16)

    mesh = plsc.VectorSubcoreMesh(core_axis_name="c", subcore_axis_name="s",
                                  num_cores=_NC, num_subcores=_NS)
    cp = pltpu.CompilerParams()
    if "needs_layout_passes" in pltpu.CompilerParams.__dataclass_fields__:
        cp = dataclasses.replace(cp, needs_layout_passes=False)
    cp = dataclasses.replace(cp, use_tc_tiling_on_sc=False)
    fm = pl.kernel(
        _fm_body,
        out_type=jax.ShapeDtypeStruct((_B,), jnp.float32),
        mesh=mesh,
        scratch_types=[
            pltpu.VMEM((_IPC,), jnp.int32),
            pltpu.VMEM((_IPC,), jnp.int32),
            pltpu.VMEM((_IPC,), jnp.int32),
            pltpu.VMEM((1664,), jnp.int32),
            pltpu.VMEM((_IPC,), jnp.float32),
            pltpu.VMEM((_IPC,), jnp.int32),
            pltpu.VMEM((_IPC,), jnp.float32),
            pltpu.VMEM((_IPC, 16), jnp.float32),
            pltpu.VMEM((_IPC, 16), jnp.float32),
            pltpu.VMEM((_G * 16,), jnp.float32),
            pltpu.VMEM((_G * 16,), jnp.float32),
            pltpu.VMEM((_EPW,), jnp.float32),
            pltpu.SemaphoreType.DMA,
            pltpu.SemaphoreType.DMA,
        ],
        compiler_params=cp,
    )
    out = fm(idxflat, valflat, offs, wt, vt)
    return out.reshape(_B, 1) + w0
